# baseline (speedup 1.0000x reference)
import functools
import math

import jax
import jax.numpy as jnp
from jax import lax
from jax.experimental import pallas as pl
from jax.experimental.pallas import tpu as pltpu
from jax.experimental.pallas import tpu_sc as plsc

B, S, D, H, E, K = 1, 2048, 768, 1536, 8, 2
T = B * S
P = T * K
MB = 128
NBLK = P // MB + E
NBLK_PAD = 48
PPAD = NBLK * MB

NW = 32
PW = P // NW
RW = PPAD // NW
RCH = 32
TW = T // NW
TCH = 64

MG = 256
TBG = T // MG

_MESH = plsc.VectorSubcoreMesh(core_axis_name="c", subcore_axis_name="s")
_SC_PARAMS = pltpu.CompilerParams(needs_layout_passes=False)


def _wid():
    return lax.axis_index("s") * 2 + lax.axis_index("c")


def _lanes():
    return lax.broadcasted_iota(jnp.int32, (16,), 0)


def _gate_body(x_ref, wg_ref, bg_ref, idx_ref, w_ref):
    xb = x_ref[...]
    s = jnp.dot(xb.astype(jnp.bfloat16), wg_ref[...].astype(jnp.bfloat16),
                preferred_element_type=jnp.float32) + bg_ref[...]
    m_, e_ = s.shape
    col = jax.lax.broadcasted_iota(jnp.int32, (m_, e_), 1)
    m1 = jnp.max(s, axis=1, keepdims=True)
    i1 = jnp.min(jnp.where(s >= m1, col, e_), axis=1, keepdims=True)
    s2 = jnp.where(col == i1, -jnp.inf, s)
    m2 = jnp.max(s2, axis=1, keepdims=True)
    i2 = jnp.min(jnp.where(s2 >= m2, col, e_), axis=1, keepdims=True)
    b = jnp.exp(m2 - m1)
    w1 = 1.0 / (1.0 + b)
    idx_ref[...] = jnp.concatenate([i1, i2], axis=1)
    w_ref[...] = jnp.concatenate([w1, 1.0 - w1], axis=1)


def _gate(x_flat, Wg, bg):
    return pl.pallas_call(
        _gate_body,
        grid=(TBG,),
        in_specs=[
            pl.BlockSpec((MG, D), lambda tb: (tb, 0)),
            pl.BlockSpec((D, E), lambda tb: (0, 0)),
            pl.BlockSpec((1, E), lambda tb: (0, 0)),
        ],
        out_specs=[
            pl.BlockSpec((MG, K), lambda tb: (tb, 0)),
            pl.BlockSpec((MG, K), lambda tb: (tb, 0)),
        ],
        out_shape=[
            jax.ShapeDtypeStruct((T, K), jnp.int32),
            jax.ShapeDtypeStruct((T, K), jnp.float32),
        ],
    )(x_flat, Wg, bg.reshape(1, E))


@functools.partial(
    pl.kernel, mesh=_MESH, compiler_params=_SC_PARAMS,
    out_type=jax.ShapeDtypeStruct((NW, 16), jnp.int32),
    scratch_types=[
        pltpu.VMEM((PW,), jnp.int32),
        pltpu.VMEM((16,), jnp.int32),
    ],
)
def _hist_kernel(e_hbm, counts_hbm, ids_v, hist_v):
    w = _wid()
    pltpu.sync_copy(e_hbm.at[pl.ds(w * PW, PW)], ids_v)
    lanes = _lanes()
    zeros16 = jnp.zeros((16,), jnp.int32)
    ones16 = jnp.ones((16,), jnp.int32)
    cnt = [zeros16] * E
    for j in range(PW // 16):
        ids = ids_v[pl.ds(j * 16, 16)]
        for e in range(E):
            m = ids == jnp.full((16,), e, jnp.int32)
            cnt[e] = cnt[e] + plsc.all_reduce_population_count(m)
    hist = zeros16
    for e in range(E):
        hist = jnp.where(lanes == jnp.full((16,), e, jnp.int32),
                         cnt[e], hist)
    hist_v[...] = hist
    pltpu.sync_copy(hist_v, counts_hbm.at[w])


@functools.partial(
    pl.kernel, mesh=_MESH, compiler_params=_SC_PARAMS,
    out_type=[
        jax.ShapeDtypeStruct((PPAD, D), jnp.float32),
        jax.ShapeDtypeStruct((PPAD,), jnp.float32),
        jax.ShapeDtypeStruct((P,), jnp.int32),
        jax.ShapeDtypeStruct((NBLK_PAD,), jnp.int32),
    ],
    scratch_types=[
        pltpu.VMEM((NW, 16), jnp.int32),
        pltpu.VMEM((PW,), jnp.int32),
        pltpu.VMEM((PW,), jnp.float32),
        pltpu.VMEM((PW,), jnp.int32),
        pltpu.VMEM((PW, D), jnp.float32),
        pltpu.VMEM((NBLK_PAD,), jnp.int32),
        pltpu.SemaphoreType.DMA,
    ],
)
def _place_kernel(e_hbm, pw_hbm, counts_hbm, x_hbm,
                  xs_hbm, wsort_hbm, pairpos_hbm, bexp_hbm,
                  counts_v, ids_v, wv_v, pos_v, xrows_v, be_v, sem):
    w = _wid()
    tbase = pl.multiple_of((w * PW) & (T - 1), PW)
    xcp = pltpu.async_copy(x_hbm.at[pl.ds(tbase, PW)], xrows_v, sem)
    pltpu.sync_copy(counts_hbm, counts_v)
    pltpu.sync_copy(e_hbm.at[pl.ds(w * PW, PW)], ids_v)
    pltpu.sync_copy(pw_hbm.at[pl.ds(w * PW, PW)], wv_v)

    def bcast(s):
        return jnp.full((16,), s, jnp.int32)

    zeros16 = jnp.zeros((16,), jnp.int32)
    ones16 = jnp.ones((16,), jnp.int32)
    lanes = _lanes()
    wv16 = bcast(w)

    totals = zeros16
    prefix = zeros16
    for wp in range(NW):
        row = counts_v[wp, :]
        totals = totals + row
        prefix = prefix + jnp.where(bcast(wp) < wv16, row, zeros16)

    cpad = ((totals + bcast(MB - 1)) // bcast(MB)) * bcast(MB)
    ends = plsc.cumsum(cpad)
    gpad = ends - cpad
    start = gpad + prefix

    cnt = [bcast(jnp.sum(jnp.where(lanes == bcast(e), start, zeros16)))
           for e in range(E)]

    for j in range(PW // 16):
        ids = ids_v[pl.ds(j * 16, 16)]
        pos = zeros16
        for e in range(E):
            m = ids == bcast(e)
            mi = jnp.where(m, ones16, zeros16)
            rank = plsc.cumsum(mi) - ones16
            pos = jnp.where(m, cnt[e] + rank, pos)
            cnt[e] = cnt[e] + plsc.all_reduce_population_count(m)
        pos_v[pl.ds(j * 16, 16)] = pos

    xcp.wait()
    pltpu.sync_copy(xrows_v, xs_hbm.at[pos_v])
    pltpu.sync_copy(wv_v, wsort_hbm.at[pos_v])
    pltpu.sync_copy(pos_v, pairpos_hbm.at[pl.ds(w * PW, PW)])

    @pl.when(w == 0)
    def _():
        nused = bcast(jnp.sum(jnp.where(lanes == bcast(E - 1), ends,
                                        zeros16))) // bcast(MB)
        for j in range(NBLK_PAD // 16):
            bstart = (bcast(j * 16) + lanes) * bcast(MB)
            be = zeros16
            for e in range(E - 1):
                end_e = bcast(jnp.sum(jnp.where(lanes == bcast(e), ends,
                                                zeros16)))
                be = be + jnp.where(bstart >= end_e, ones16, zeros16)
            if j == NBLK_PAD // 16 - 1:
                be = jnp.where(lanes == bcast(15), nused, be)
            be_v[pl.ds(j * 16, 16)] = be
        pltpu.sync_copy(be_v, bexp_hbm)


def _gelu_exact(v):
    return 0.5 * v * (1.0 + jax.lax.erf(v * (1.0 / math.sqrt(2.0))))


def _mlp_body(be_ref, xs_ref, w1_ref, b1_ref, w2_ref, b2_ref, ws_ref, y_ref,
              w1bf_s, w2bf_s):
    b = pl.program_id(0)
    nused = be_ref[NBLK_PAD - 1]

    new_expert = jnp.logical_or(
        b == 0, be_ref[b] != be_ref[jnp.maximum(b - 1, 0)])

    @pl.when(jnp.logical_and(b < nused, new_expert))
    def _():
        w1bf_s[...] = w1_ref[0].astype(jnp.bfloat16)
        w2bf_s[...] = w2_ref[0].astype(jnp.bfloat16)

    @pl.when(b < nused)
    def _():
        xb = xs_ref[...].astype(jnp.bfloat16)
        h = jnp.dot(xb, w1bf_s[...], preferred_element_type=jnp.float32)
        h = _gelu_exact(h + b1_ref[0])
        y = jnp.dot(h.astype(jnp.bfloat16), w2bf_s[...],
                    preferred_element_type=jnp.float32) + b2_ref[0]
        y_ref[...] = ws_ref[...] * y


def _grouped_mlp(block_expert, Xs, W1bf, b1, W2bf, b2, w_sorted):
    grid_spec = pltpu.PrefetchScalarGridSpec(
        num_scalar_prefetch=1,
        grid=(NBLK,),
        in_specs=[
            pl.BlockSpec((MB, D), lambda b, be: (b, 0)),
            pl.BlockSpec((1, D, H), lambda b, be: (be[b], 0, 0)),
            pl.BlockSpec((1, 1, H), lambda b, be: (be[b], 0, 0)),
            pl.BlockSpec((1, H, D), lambda b, be: (be[b], 0, 0)),
            pl.BlockSpec((1, 1, D), lambda b, be: (be[b], 0, 0)),
            pl.BlockSpec((MB, 1), lambda b, be: (b, 0)),
        ],
        out_specs=pl.BlockSpec((MB, D), lambda b, be: (b, 0)),
        scratch_shapes=[
            pltpu.VMEM((D, H), jnp.bfloat16),
            pltpu.VMEM((H, D), jnp.bfloat16),
        ],
    )
    return pl.pallas_call(
        _mlp_body,
        grid_spec=grid_spec,
        out_shape=jax.ShapeDtypeStruct((PPAD, D), jnp.float32),
    )(block_expert, Xs, W1bf, b1.reshape(E, 1, H), W2bf,
      b2.reshape(E, 1, D), w_sorted.reshape(PPAD, 1))


@functools.partial(
    pl.kernel, mesh=_MESH, compiler_params=_SC_PARAMS,
    out_type=jax.ShapeDtypeStruct((T, D), jnp.float32),
    scratch_types=[
        pltpu.VMEM((TCH,), jnp.int32),
        pltpu.VMEM((TCH,), jnp.int32),
        pltpu.VMEM((TCH, D), jnp.float32),
        pltpu.VMEM((TCH, D), jnp.float32),
        pltpu.SemaphoreType.DMA,
        pltpu.SemaphoreType.DMA,
    ],
)
def _combine_kernel(y_hbm, pairpos_hbm, out_hbm, idx0_v, idx1_v, buf0, buf1,
                    sem0, sem1):
    w = _wid()
    for c in range(TW // TCH):
        tbase = w * TW + c * TCH
        pltpu.sync_copy(pairpos_hbm.at[pl.ds(tbase, TCH)], idx0_v)
        pltpu.sync_copy(pairpos_hbm.at[pl.ds(T + tbase, TCH)], idx1_v)
        cp0 = pltpu.async_copy(y_hbm.at[idx0_v], buf0, sem0)
        cp1 = pltpu.async_copy(y_hbm.at[idx1_v], buf1, sem1)
        cp0.wait()
        cp1.wait()

        def add_row(i, carry):
            for cc in range(D // 16):
                a = buf0[i, pl.ds(cc * 16, 16)]
                b = buf1[i, pl.ds(cc * 16, 16)]
                buf0[i, pl.ds(cc * 16, 16)] = a + b
            return carry

        lax.fori_loop(0, TCH, add_row, 0)
        pltpu.sync_copy(buf0, out_hbm.at[pl.ds(tbase, TCH)])


@jax.jit
def kernel(x, Wg, bg, W1, b1, W2, b2):
    x_flat = x.reshape(T, D)
    idx, pw = _gate(x_flat, Wg, bg)
    e_flat = idx.T.reshape(P)
    pw_flat = pw.T.reshape(P)

    counts = _hist_kernel(e_flat)
    Xs, w_sorted, pair_pos, block_expert = _place_kernel(
        e_flat, pw_flat, counts, x_flat)
    Y = _grouped_mlp(block_expert, Xs, W1, b1, W2, b2, w_sorted)
    out = _combine_kernel(Y, pair_pos)
    return out.reshape(B, S, D)


def _jnp_route(e_flat, pw_flat):
    c = jnp.bincount(e_flat, length=E)
    cpad = ((c + MB - 1) // MB) * MB
    ends = jnp.cumsum(cpad)
    gpad = ends - cpad
    start = jnp.cumsum(c) - c
    order = jnp.argsort(e_flat, stable=True)
    es = e_flat[order]
    pos_sorted = gpad[es] + (jnp.arange(P, dtype=jnp.int32) - start[es])
    pair_pos = jnp.zeros((P,), jnp.int32).at[order].set(pos_sorted)
    src_token = jnp.zeros((PPAD,), jnp.int32).at[pos_sorted].set(
        order.astype(jnp.int32) & (T - 1))
    w_sorted = jnp.zeros((PPAD,), jnp.float32).at[pos_sorted].set(
        pw_flat[order])
    block_expert = jnp.minimum(
        jnp.searchsorted(ends, jnp.arange(NBLK, dtype=jnp.int32) * MB,
                         side="right"), E - 1).astype(jnp.int32)
    return src_token, w_sorted, pair_pos, block_expert


# speedup vs baseline: 1.0642x; 1.0642x over previous
import functools
import math

import jax
import jax.numpy as jnp
from jax import lax
from jax.experimental import pallas as pl
from jax.experimental.pallas import tpu as pltpu
from jax.experimental.pallas import tpu_sc as plsc

B, S, D, H, E, K = 1, 2048, 768, 1536, 8, 2
T = B * S
P = T * K
MB = 256
NBLK = P // MB + E
NBLK_PAD = 48
PPAD = NBLK * MB

NW = 32
PW = P // NW
RW = PPAD // NW
RCH = 32
TW = T // NW
TCH = 64

MG = 256
TBG = T // MG

_MESH = plsc.VectorSubcoreMesh(core_axis_name="c", subcore_axis_name="s")
_SC_PARAMS = pltpu.CompilerParams(needs_layout_passes=False)


def _wid():
    return lax.axis_index("s") * 2 + lax.axis_index("c")


def _lanes():
    return lax.broadcasted_iota(jnp.int32, (16,), 0)


def _gate_body(x_ref, wg_ref, bg_ref, idx_ref, w_ref):
    xb = x_ref[...]
    s = jnp.dot(xb.astype(jnp.bfloat16), wg_ref[...].astype(jnp.bfloat16),
                preferred_element_type=jnp.float32) + bg_ref[...]
    m_, e_ = s.shape
    col = jax.lax.broadcasted_iota(jnp.int32, (m_, e_), 1)
    m1 = jnp.max(s, axis=1, keepdims=True)
    i1 = jnp.min(jnp.where(s >= m1, col, e_), axis=1, keepdims=True)
    s2 = jnp.where(col == i1, -jnp.inf, s)
    m2 = jnp.max(s2, axis=1, keepdims=True)
    i2 = jnp.min(jnp.where(s2 >= m2, col, e_), axis=1, keepdims=True)
    b = jnp.exp(m2 - m1)
    w1 = 1.0 / (1.0 + b)
    idx_ref[...] = jnp.concatenate([i1, i2], axis=1)
    w_ref[...] = jnp.concatenate([w1, 1.0 - w1], axis=1)


def _gate(x_flat, Wg, bg):
    return pl.pallas_call(
        _gate_body,
        grid=(TBG,),
        in_specs=[
            pl.BlockSpec((MG, D), lambda tb: (tb, 0)),
            pl.BlockSpec((D, E), lambda tb: (0, 0)),
            pl.BlockSpec((1, E), lambda tb: (0, 0)),
        ],
        out_specs=[
            pl.BlockSpec((MG, K), lambda tb: (tb, 0)),
            pl.BlockSpec((MG, K), lambda tb: (tb, 0)),
        ],
        out_shape=[
            jax.ShapeDtypeStruct((T, K), jnp.int32),
            jax.ShapeDtypeStruct((T, K), jnp.float32),
        ],
    )(x_flat, Wg, bg.reshape(1, E))


@functools.partial(
    pl.kernel, mesh=_MESH, compiler_params=_SC_PARAMS,
    out_type=jax.ShapeDtypeStruct((NW, 16), jnp.int32),
    scratch_types=[
        pltpu.VMEM((PW,), jnp.int32),
        pltpu.VMEM((16,), jnp.int32),
    ],
)
def _hist_kernel(e_hbm, counts_hbm, ids_v, hist_v):
    w = _wid()
    pltpu.sync_copy(e_hbm.at[pl.ds(w * PW, PW)], ids_v)
    lanes = _lanes()
    zeros16 = jnp.zeros((16,), jnp.int32)
    ones16 = jnp.ones((16,), jnp.int32)
    cnt = [zeros16] * E
    for j in range(PW // 16):
        ids = ids_v[pl.ds(j * 16, 16)]
        for e in range(E):
            m = ids == jnp.full((16,), e, jnp.int32)
            cnt[e] = cnt[e] + plsc.all_reduce_population_count(m)
    hist = zeros16
    for e in range(E):
        hist = jnp.where(lanes == jnp.full((16,), e, jnp.int32),
                         cnt[e], hist)
    hist_v[...] = hist
    pltpu.sync_copy(hist_v, counts_hbm.at[w])


@functools.partial(
    pl.kernel, mesh=_MESH, compiler_params=_SC_PARAMS,
    out_type=[
        jax.ShapeDtypeStruct((PPAD, D), jnp.float32),
        jax.ShapeDtypeStruct((PPAD,), jnp.float32),
        jax.ShapeDtypeStruct((P,), jnp.int32),
        jax.ShapeDtypeStruct((NBLK_PAD,), jnp.int32),
    ],
    scratch_types=[
        pltpu.VMEM((NW, 16), jnp.int32),
        pltpu.VMEM((PW,), jnp.int32),
        pltpu.VMEM((PW,), jnp.float32),
        pltpu.VMEM((PW,), jnp.int32),
        pltpu.VMEM((PW, D), jnp.float32),
        pltpu.VMEM((NBLK_PAD,), jnp.int32),
        pltpu.SemaphoreType.DMA,
    ],
)
def _place_kernel(e_hbm, pw_hbm, counts_hbm, x_hbm,
                  xs_hbm, wsort_hbm, pairpos_hbm, bexp_hbm,
                  counts_v, ids_v, wv_v, pos_v, xrows_v, be_v, sem):
    w = _wid()
    tbase = pl.multiple_of((w * PW) & (T - 1), PW)
    xcp = pltpu.async_copy(x_hbm.at[pl.ds(tbase, PW)], xrows_v, sem)
    pltpu.sync_copy(counts_hbm, counts_v)
    pltpu.sync_copy(e_hbm.at[pl.ds(w * PW, PW)], ids_v)
    pltpu.sync_copy(pw_hbm.at[pl.ds(w * PW, PW)], wv_v)

    def bcast(s):
        return jnp.full((16,), s, jnp.int32)

    zeros16 = jnp.zeros((16,), jnp.int32)
    ones16 = jnp.ones((16,), jnp.int32)
    lanes = _lanes()
    wv16 = bcast(w)

    totals = zeros16
    prefix = zeros16
    for wp in range(NW):
        row = counts_v[wp, :]
        totals = totals + row
        prefix = prefix + jnp.where(bcast(wp) < wv16, row, zeros16)

    cpad = ((totals + bcast(MB - 1)) // bcast(MB)) * bcast(MB)
    ends = plsc.cumsum(cpad)
    gpad = ends - cpad
    start = gpad + prefix

    cnt = [bcast(jnp.sum(jnp.where(lanes == bcast(e), start, zeros16)))
           for e in range(E)]

    for j in range(PW // 16):
        ids = ids_v[pl.ds(j * 16, 16)]
        pos = zeros16
        for e in range(E):
            m = ids == bcast(e)
            mi = jnp.where(m, ones16, zeros16)
            rank = plsc.cumsum(mi) - ones16
            pos = jnp.where(m, cnt[e] + rank, pos)
            cnt[e] = cnt[e] + plsc.all_reduce_population_count(m)
        pos_v[pl.ds(j * 16, 16)] = pos

    xcp.wait()
    pltpu.sync_copy(xrows_v, xs_hbm.at[pos_v])
    pltpu.sync_copy(wv_v, wsort_hbm.at[pos_v])
    pltpu.sync_copy(pos_v, pairpos_hbm.at[pl.ds(w * PW, PW)])

    @pl.when(w == 0)
    def _():
        nused = bcast(jnp.sum(jnp.where(lanes == bcast(E - 1), ends,
                                        zeros16))) // bcast(MB)
        for j in range(NBLK_PAD // 16):
            bstart = (bcast(j * 16) + lanes) * bcast(MB)
            be = zeros16
            for e in range(E - 1):
                end_e = bcast(jnp.sum(jnp.where(lanes == bcast(e), ends,
                                                zeros16)))
                be = be + jnp.where(bstart >= end_e, ones16, zeros16)
            if j == NBLK_PAD // 16 - 1:
                be = jnp.where(lanes == bcast(15), nused, be)
            be_v[pl.ds(j * 16, 16)] = be
        pltpu.sync_copy(be_v, bexp_hbm)


def _gelu_exact(v):
    return 0.5 * v * (1.0 + jax.lax.erf(v * (1.0 / math.sqrt(2.0))))


def _mlp_body(be_ref, xs_ref, w1_ref, b1_ref, w2_ref, b2_ref, ws_ref, y_ref,
              w1bf_s, w2bf_s):
    b = pl.program_id(0)
    nused = be_ref[NBLK_PAD - 1]

    new_expert = jnp.logical_or(
        b == 0, be_ref[b] != be_ref[jnp.maximum(b - 1, 0)])

    @pl.when(jnp.logical_and(b < nused, new_expert))
    def _():
        w1bf_s[...] = w1_ref[0].astype(jnp.bfloat16)
        w2bf_s[...] = w2_ref[0].astype(jnp.bfloat16)

    @pl.when(b < nused)
    def _():
        xb = xs_ref[...].astype(jnp.bfloat16)
        h = jnp.dot(xb, w1bf_s[...], preferred_element_type=jnp.float32)
        h = _gelu_exact(h + b1_ref[0])
        y = jnp.dot(h.astype(jnp.bfloat16), w2bf_s[...],
                    preferred_element_type=jnp.float32) + b2_ref[0]
        y_ref[...] = ws_ref[...] * y


def _grouped_mlp(block_expert, Xs, W1bf, b1, W2bf, b2, w_sorted):
    grid_spec = pltpu.PrefetchScalarGridSpec(
        num_scalar_prefetch=1,
        grid=(NBLK,),
        in_specs=[
            pl.BlockSpec((MB, D), lambda b, be: (b, 0)),
            pl.BlockSpec((1, D, H), lambda b, be: (be[b], 0, 0)),
            pl.BlockSpec((1, 1, H), lambda b, be: (be[b], 0, 0)),
            pl.BlockSpec((1, H, D), lambda b, be: (be[b], 0, 0)),
            pl.BlockSpec((1, 1, D), lambda b, be: (be[b], 0, 0)),
            pl.BlockSpec((MB, 1), lambda b, be: (b, 0)),
        ],
        out_specs=pl.BlockSpec((MB, D), lambda b, be: (b, 0)),
        scratch_shapes=[
            pltpu.VMEM((D, H), jnp.bfloat16),
            pltpu.VMEM((H, D), jnp.bfloat16),
        ],
    )
    return pl.pallas_call(
        _mlp_body,
        grid_spec=grid_spec,
        out_shape=jax.ShapeDtypeStruct((PPAD, D), jnp.float32),
    )(block_expert, Xs, W1bf, b1.reshape(E, 1, H), W2bf,
      b2.reshape(E, 1, D), w_sorted.reshape(PPAD, 1))


@functools.partial(
    pl.kernel, mesh=_MESH, compiler_params=_SC_PARAMS,
    out_type=jax.ShapeDtypeStruct((T, D), jnp.float32),
    scratch_types=[
        pltpu.VMEM((TCH,), jnp.int32),
        pltpu.VMEM((TCH,), jnp.int32),
        pltpu.VMEM((TCH, D), jnp.float32),
        pltpu.VMEM((TCH, D), jnp.float32),
        pltpu.SemaphoreType.DMA,
        pltpu.SemaphoreType.DMA,
    ],
)
def _combine_kernel(y_hbm, pairpos_hbm, out_hbm, idx0_v, idx1_v, buf0, buf1,
                    sem0, sem1):
    w = _wid()
    for c in range(TW // TCH):
        tbase = w * TW + c * TCH
        pltpu.sync_copy(pairpos_hbm.at[pl.ds(tbase, TCH)], idx0_v)
        pltpu.sync_copy(pairpos_hbm.at[pl.ds(T + tbase, TCH)], idx1_v)
        cp0 = pltpu.async_copy(y_hbm.at[idx0_v], buf0, sem0)
        cp1 = pltpu.async_copy(y_hbm.at[idx1_v], buf1, sem1)
        cp0.wait()
        cp1.wait()

        def add_row(i, carry):
            for cc in range(D // 16):
                a = buf0[i, pl.ds(cc * 16, 16)]
                b = buf1[i, pl.ds(cc * 16, 16)]
                buf0[i, pl.ds(cc * 16, 16)] = a + b
            return carry

        lax.fori_loop(0, TCH, add_row, 0)
        pltpu.sync_copy(buf0, out_hbm.at[pl.ds(tbase, TCH)])


@jax.jit
def kernel(x, Wg, bg, W1, b1, W2, b2):
    x_flat = x.reshape(T, D)
    idx, pw = _gate(x_flat, Wg, bg)
    e_flat = idx.T.reshape(P)
    pw_flat = pw.T.reshape(P)

    counts = _hist_kernel(e_flat)
    Xs, w_sorted, pair_pos, block_expert = _place_kernel(
        e_flat, pw_flat, counts, x_flat)
    Y = _grouped_mlp(block_expert, Xs, W1, b1, W2, b2, w_sorted)
    out = _combine_kernel(Y, pair_pos)
    return out.reshape(B, S, D)


def _jnp_route(e_flat, pw_flat):
    c = jnp.bincount(e_flat, length=E)
    cpad = ((c + MB - 1) // MB) * MB
    ends = jnp.cumsum(cpad)
    gpad = ends - cpad
    start = jnp.cumsum(c) - c
    order = jnp.argsort(e_flat, stable=True)
    es = e_flat[order]
    pos_sorted = gpad[es] + (jnp.arange(P, dtype=jnp.int32) - start[es])
    pair_pos = jnp.zeros((P,), jnp.int32).at[order].set(pos_sorted)
    src_token = jnp.zeros((PPAD,), jnp.int32).at[pos_sorted].set(
        order.astype(jnp.int32) & (T - 1))
    w_sorted = jnp.zeros((PPAD,), jnp.float32).at[pos_sorted].set(
        pw_flat[order])
    block_expert = jnp.minimum(
        jnp.searchsorted(ends, jnp.arange(NBLK, dtype=jnp.int32) * MB,
                         side="right"), E - 1).astype(jnp.int32)
    return src_token, w_sorted, pair_pos, block_expert


# speedup vs baseline: 1.1441x; 1.0750x over previous
import functools
import math

import jax
import jax.numpy as jnp
from jax import lax
from jax.experimental import pallas as pl
from jax.experimental.pallas import tpu as pltpu
from jax.experimental.pallas import tpu_sc as plsc

B, S, D, H, E, K = 1, 2048, 768, 1536, 8, 2
T = B * S
P = T * K
MB = 256
NBLK = P // MB + E
NBLK_PAD = 48
PPAD = NBLK * MB

NW = 32
PW = P // NW
RW = PPAD // NW
RCH = 32
TW = T // NW
TCH = 64

MG = 256
TBG = T // MG

_MESH = plsc.VectorSubcoreMesh(core_axis_name="c", subcore_axis_name="s")
_SC_PARAMS = pltpu.CompilerParams(needs_layout_passes=False)


def _wid():
    return lax.axis_index("s") * 2 + lax.axis_index("c")


def _lanes():
    return lax.broadcasted_iota(jnp.int32, (16,), 0)


def _gate_body(x_ref, wg_ref, bg_ref, idx_ref, w_ref):
    xb = x_ref[...]
    s = jnp.dot(xb.astype(jnp.bfloat16), wg_ref[...].astype(jnp.bfloat16),
                preferred_element_type=jnp.float32) + bg_ref[...]
    m_, e_ = s.shape
    col = jax.lax.broadcasted_iota(jnp.int32, (m_, e_), 1)
    m1 = jnp.max(s, axis=1, keepdims=True)
    i1 = jnp.min(jnp.where(s >= m1, col, e_), axis=1, keepdims=True)
    s2 = jnp.where(col == i1, -jnp.inf, s)
    m2 = jnp.max(s2, axis=1, keepdims=True)
    i2 = jnp.min(jnp.where(s2 >= m2, col, e_), axis=1, keepdims=True)
    b = jnp.exp(m2 - m1)
    w1 = 1.0 / (1.0 + b)
    idx_ref[...] = jnp.concatenate([i1, i2], axis=1)
    w_ref[...] = jnp.concatenate([w1, 1.0 - w1], axis=1)


def _gate(x_flat, Wg, bg):
    return pl.pallas_call(
        _gate_body,
        grid=(TBG,),
        in_specs=[
            pl.BlockSpec((MG, D), lambda tb: (tb, 0)),
            pl.BlockSpec((D, E), lambda tb: (0, 0)),
            pl.BlockSpec((1, E), lambda tb: (0, 0)),
        ],
        out_specs=[
            pl.BlockSpec((MG, K), lambda tb: (tb, 0)),
            pl.BlockSpec((MG, K), lambda tb: (tb, 0)),
        ],
        out_shape=[
            jax.ShapeDtypeStruct((T, K), jnp.int32),
            jax.ShapeDtypeStruct((T, K), jnp.float32),
        ],
    )(x_flat, Wg, bg.reshape(1, E))


@functools.partial(
    pl.kernel, mesh=_MESH, compiler_params=_SC_PARAMS,
    out_type=jax.ShapeDtypeStruct((NW, 16), jnp.int32),
    scratch_types=[
        pltpu.VMEM((PW,), jnp.int32),
        pltpu.VMEM((16,), jnp.int32),
    ],
)
def _hist_kernel(e_hbm, counts_hbm, ids_v, hist_v):
    w = _wid()
    pltpu.sync_copy(e_hbm.at[pl.ds(w * PW, PW)], ids_v)
    lanes = _lanes()
    zeros16 = jnp.zeros((16,), jnp.int32)
    ones16 = jnp.ones((16,), jnp.int32)
    cnt = [zeros16] * E
    for j in range(PW // 16):
        ids = ids_v[pl.ds(j * 16, 16)]
        for e in range(E):
            m = ids == jnp.full((16,), e, jnp.int32)
            cnt[e] = cnt[e] + plsc.all_reduce_population_count(m)
    hist = zeros16
    for e in range(E):
        hist = jnp.where(lanes == jnp.full((16,), e, jnp.int32),
                         cnt[e], hist)
    hist_v[...] = hist
    pltpu.sync_copy(hist_v, counts_hbm.at[w])


@functools.partial(
    pl.kernel, mesh=_MESH, compiler_params=_SC_PARAMS,
    out_type=[
        jax.ShapeDtypeStruct((PPAD, D), jnp.float32),
        jax.ShapeDtypeStruct((PPAD,), jnp.float32),
        jax.ShapeDtypeStruct((P,), jnp.int32),
        jax.ShapeDtypeStruct((NBLK_PAD,), jnp.int32),
    ],
    scratch_types=[
        pltpu.VMEM((NW, 16), jnp.int32),
        pltpu.VMEM((PW,), jnp.int32),
        pltpu.VMEM((PW,), jnp.float32),
        pltpu.VMEM((PW,), jnp.int32),
        pltpu.VMEM((PW, D), jnp.float32),
        pltpu.VMEM((NBLK_PAD,), jnp.int32),
        pltpu.SemaphoreType.DMA,
    ],
)
def _place_kernel(e_hbm, pw_hbm, counts_hbm, x_hbm,
                  xs_hbm, wsort_hbm, pairpos_hbm, bexp_hbm,
                  counts_v, ids_v, wv_v, pos_v, xrows_v, be_v, sem):
    w = _wid()
    tbase = pl.multiple_of((w * PW) & (T - 1), PW)
    xcp = pltpu.async_copy(x_hbm.at[pl.ds(tbase, PW)], xrows_v, sem)
    pltpu.sync_copy(counts_hbm, counts_v)
    pltpu.sync_copy(e_hbm.at[pl.ds(w * PW, PW)], ids_v)
    pltpu.sync_copy(pw_hbm.at[pl.ds(w * PW, PW)], wv_v)

    def bcast(s):
        return jnp.full((16,), s, jnp.int32)

    zeros16 = jnp.zeros((16,), jnp.int32)
    ones16 = jnp.ones((16,), jnp.int32)
    lanes = _lanes()
    wv16 = bcast(w)

    totals = zeros16
    prefix = zeros16
    for wp in range(NW):
        row = counts_v[wp, :]
        totals = totals + row
        prefix = prefix + jnp.where(bcast(wp) < wv16, row, zeros16)

    cpad = ((totals + bcast(MB - 1)) // bcast(MB)) * bcast(MB)
    ends = plsc.cumsum(cpad)
    gpad = ends - cpad
    start = gpad + prefix

    cnt = [bcast(jnp.sum(jnp.where(lanes == bcast(e), start, zeros16)))
           for e in range(E)]

    for j in range(PW // 16):
        ids = ids_v[pl.ds(j * 16, 16)]
        pos = zeros16
        for e in range(E):
            m = ids == bcast(e)
            mi = jnp.where(m, ones16, zeros16)
            rank = plsc.cumsum(mi) - ones16
            pos = jnp.where(m, cnt[e] + rank, pos)
            cnt[e] = cnt[e] + plsc.all_reduce_population_count(m)
        pos_v[pl.ds(j * 16, 16)] = pos

    xcp.wait()
    pltpu.sync_copy(xrows_v, xs_hbm.at[pos_v])
    pltpu.sync_copy(wv_v, wsort_hbm.at[pos_v])
    pltpu.sync_copy(pos_v, pairpos_hbm.at[pl.ds(w * PW, PW)])

    @pl.when(w == 0)
    def _():
        nused = bcast(jnp.sum(jnp.where(lanes == bcast(E - 1), ends,
                                        zeros16))) // bcast(MB)
        for j in range(NBLK_PAD // 16):
            bstart = (bcast(j * 16) + lanes) * bcast(MB)
            be = zeros16
            for e in range(E - 1):
                end_e = bcast(jnp.sum(jnp.where(lanes == bcast(e), ends,
                                                zeros16)))
                be = be + jnp.where(bstart >= end_e, ones16, zeros16)
            if j == NBLK_PAD // 16 - 1:
                be = jnp.where(lanes == bcast(15), nused, be)
            be_v[pl.ds(j * 16, 16)] = be
        pltpu.sync_copy(be_v, bexp_hbm)


def _gelu_exact(v):
    return 0.5 * v * (1.0 + jax.lax.erf(v * (1.0 / math.sqrt(2.0))))


def _mlp_body(be_ref, xs_ref, w1_ref, b1_ref, w2_ref, b2_ref, ws_ref, y_ref,
              w1f_s, w2f_s, w1bf_s, w2bf_s, sems):
    b = pl.program_id(0)
    nused = be_ref[NBLK_PAD - 1]

    isb = []
    ords = []
    o = jnp.int32(-1)
    for bp in range(NBLK):
        first = jnp.logical_and(
            bp < nused,
            (be_ref[bp] != be_ref[bp - 1]) if bp else jnp.bool_(True))
        isb.append(first)
        o = o + first.astype(jnp.int32)
        ords.append(o)

    def sel(vals, idx):
        acc = jnp.int32(0)
        for bp in range(NBLK):
            acc = jnp.where(idx == bp, vals[bp].astype(jnp.int32), acc)
        return acc

    ordb = sel(ords, b)
    new_exp = sel(isb, b) > 0

    def group_expert(m):
        ge = jnp.int32(0)
        ex = jnp.bool_(False)
        for bp in range(NBLK):
            hit = jnp.logical_and(isb[bp], ords[bp] == m)
            ge = jnp.where(hit, be_ref[bp], ge)
            ex = jnp.logical_or(ex, hit)
        return ge, ex

    def fetch(m, slot):
        ge, ex = group_expert(m)

        @pl.when(ex)
        def _():
            pltpu.make_async_copy(w1_ref.at[ge], w1f_s.at[slot],
                                  sems.at[slot]).start()
            pltpu.make_async_copy(w2_ref.at[ge], w2f_s.at[slot],
                                  sems.at[slot]).start()

    @pl.when(b == 0)
    def _():
        fetch(jnp.int32(0), jnp.int32(0))
        fetch(jnp.int32(1), jnp.int32(1))

    @pl.when(jnp.logical_and(b < nused, new_exp))
    def _():
        slot = jnp.remainder(ordb, 2)
        e_cur = be_ref[b]
        pltpu.make_async_copy(w1_ref.at[e_cur], w1f_s.at[slot],
                              sems.at[slot]).wait()
        pltpu.make_async_copy(w2_ref.at[e_cur], w2f_s.at[slot],
                              sems.at[slot]).wait()
        w1bf_s[...] = w1f_s[slot].astype(jnp.bfloat16)
        w2bf_s[...] = w2f_s[slot].astype(jnp.bfloat16)
        fetch(ordb + 2, slot)

    @pl.when(b < nused)
    def _():
        xb = xs_ref[...].astype(jnp.bfloat16)
        h = jnp.dot(xb, w1bf_s[...], preferred_element_type=jnp.float32)
        h = _gelu_exact(h + b1_ref[0])
        y = jnp.dot(h.astype(jnp.bfloat16), w2bf_s[...],
                    preferred_element_type=jnp.float32) + b2_ref[0]
        y_ref[...] = ws_ref[...] * y


def _grouped_mlp(block_expert, Xs, W1, b1, W2, b2, w_sorted):
    grid_spec = pltpu.PrefetchScalarGridSpec(
        num_scalar_prefetch=1,
        grid=(NBLK,),
        in_specs=[
            pl.BlockSpec((MB, D), lambda b, be: (b, 0)),
            pl.BlockSpec(memory_space=pl.ANY),
            pl.BlockSpec((1, 1, H), lambda b, be: (be[b], 0, 0)),
            pl.BlockSpec(memory_space=pl.ANY),
            pl.BlockSpec((1, 1, D), lambda b, be: (be[b], 0, 0)),
            pl.BlockSpec((MB, 1), lambda b, be: (b, 0)),
        ],
        out_specs=pl.BlockSpec((MB, D), lambda b, be: (b, 0)),
        scratch_shapes=[
            pltpu.VMEM((2, D, H), jnp.float32),
            pltpu.VMEM((2, H, D), jnp.float32),
            pltpu.VMEM((D, H), jnp.bfloat16),
            pltpu.VMEM((H, D), jnp.bfloat16),
            pltpu.SemaphoreType.DMA((2,)),
        ],
    )
    return pl.pallas_call(
        _mlp_body,
        grid_spec=grid_spec,
        out_shape=jax.ShapeDtypeStruct((PPAD, D), jnp.float32),
    )(block_expert, Xs, W1, b1.reshape(E, 1, H), W2,
      b2.reshape(E, 1, D), w_sorted.reshape(PPAD, 1))


@functools.partial(
    pl.kernel, mesh=_MESH, compiler_params=_SC_PARAMS,
    out_type=jax.ShapeDtypeStruct((T, D), jnp.float32),
    scratch_types=[
        pltpu.VMEM((TCH,), jnp.int32),
        pltpu.VMEM((TCH,), jnp.int32),
        pltpu.VMEM((TCH, D), jnp.float32),
        pltpu.VMEM((TCH, D), jnp.float32),
        pltpu.SemaphoreType.DMA,
        pltpu.SemaphoreType.DMA,
    ],
)
def _combine_kernel(y_hbm, pairpos_hbm, out_hbm, idx0_v, idx1_v, buf0, buf1,
                    sem0, sem1):
    w = _wid()
    for c in range(TW // TCH):
        tbase = w * TW + c * TCH
        pltpu.sync_copy(pairpos_hbm.at[pl.ds(tbase, TCH)], idx0_v)
        pltpu.sync_copy(pairpos_hbm.at[pl.ds(T + tbase, TCH)], idx1_v)
        cp0 = pltpu.async_copy(y_hbm.at[idx0_v], buf0, sem0)
        cp1 = pltpu.async_copy(y_hbm.at[idx1_v], buf1, sem1)
        cp0.wait()
        cp1.wait()

        def add_row(i, carry):
            for cc in range(D // 16):
                a = buf0[i, pl.ds(cc * 16, 16)]
                b = buf1[i, pl.ds(cc * 16, 16)]
                buf0[i, pl.ds(cc * 16, 16)] = a + b
            return carry

        lax.fori_loop(0, TCH, add_row, 0)
        pltpu.sync_copy(buf0, out_hbm.at[pl.ds(tbase, TCH)])


@jax.jit
def kernel(x, Wg, bg, W1, b1, W2, b2):
    x_flat = x.reshape(T, D)
    idx, pw = _gate(x_flat, Wg, bg)
    e_flat = idx.T.reshape(P)
    pw_flat = pw.T.reshape(P)

    counts = _hist_kernel(e_flat)
    Xs, w_sorted, pair_pos, block_expert = _place_kernel(
        e_flat, pw_flat, counts, x_flat)
    Y = _grouped_mlp(block_expert, Xs, W1, b1, W2, b2, w_sorted)
    out = _combine_kernel(Y, pair_pos)
    return out.reshape(B, S, D)


def _jnp_route(e_flat, pw_flat):
    c = jnp.bincount(e_flat, length=E)
    cpad = ((c + MB - 1) // MB) * MB
    ends = jnp.cumsum(cpad)
    gpad = ends - cpad
    start = jnp.cumsum(c) - c
    order = jnp.argsort(e_flat, stable=True)
    es = e_flat[order]
    pos_sorted = gpad[es] + (jnp.arange(P, dtype=jnp.int32) - start[es])
    pair_pos = jnp.zeros((P,), jnp.int32).at[order].set(pos_sorted)
    src_token = jnp.zeros((PPAD,), jnp.int32).at[pos_sorted].set(
        order.astype(jnp.int32) & (T - 1))
    w_sorted = jnp.zeros((PPAD,), jnp.float32).at[pos_sorted].set(
        pw_flat[order])
    block_expert = jnp.minimum(
        jnp.searchsorted(ends, jnp.arange(NBLK, dtype=jnp.int32) * MB,
                         side="right"), E - 1).astype(jnp.int32)
    return src_token, w_sorted, pair_pos, block_expert


# speedup vs baseline: 1.1663x; 1.0195x over previous
import functools
import math

import jax
import jax.numpy as jnp
from jax import lax
from jax.experimental import pallas as pl
from jax.experimental.pallas import tpu as pltpu
from jax.experimental.pallas import tpu_sc as plsc

B, S, D, H, E, K = 1, 2048, 768, 1536, 8, 2
T = B * S
P = T * K
MB = 256
NBLK = P // MB + E
NBLK_PAD = 48
PPAD = NBLK * MB

NW = 32
PW = P // NW
RW = PPAD // NW
RCH = 32
TW = T // NW
TCH = 64

MG = 256
TBG = T // MG

_MESH = plsc.VectorSubcoreMesh(core_axis_name="c", subcore_axis_name="s")
_SC_PARAMS = pltpu.CompilerParams(needs_layout_passes=False)


def _wid():
    return lax.axis_index("s") * 2 + lax.axis_index("c")


def _lanes():
    return lax.broadcasted_iota(jnp.int32, (16,), 0)


def _gate_body(x_ref, wg_ref, bg_ref, idx_ref, w_ref):
    xb = x_ref[...]
    s = jnp.dot(xb.astype(jnp.bfloat16), wg_ref[...].astype(jnp.bfloat16),
                preferred_element_type=jnp.float32) + bg_ref[...]
    m_, e_ = s.shape
    col = jax.lax.broadcasted_iota(jnp.int32, (m_, e_), 1)
    m1 = jnp.max(s, axis=1, keepdims=True)
    i1 = jnp.min(jnp.where(s >= m1, col, e_), axis=1, keepdims=True)
    s2 = jnp.where(col == i1, -jnp.inf, s)
    m2 = jnp.max(s2, axis=1, keepdims=True)
    i2 = jnp.min(jnp.where(s2 >= m2, col, e_), axis=1, keepdims=True)
    b = jnp.exp(m2 - m1)
    w1 = 1.0 / (1.0 + b)
    idx_ref[...] = jnp.concatenate([i1, i2], axis=1)
    w_ref[...] = jnp.concatenate([w1, 1.0 - w1], axis=1)


def _gate(x_flat, Wg, bg):
    return pl.pallas_call(
        _gate_body,
        grid=(TBG,),
        in_specs=[
            pl.BlockSpec((MG, D), lambda tb: (tb, 0)),
            pl.BlockSpec((D, E), lambda tb: (0, 0)),
            pl.BlockSpec((1, E), lambda tb: (0, 0)),
        ],
        out_specs=[
            pl.BlockSpec((MG, K), lambda tb: (tb, 0)),
            pl.BlockSpec((MG, K), lambda tb: (tb, 0)),
        ],
        out_shape=[
            jax.ShapeDtypeStruct((T, K), jnp.int32),
            jax.ShapeDtypeStruct((T, K), jnp.float32),
        ],
    )(x_flat, Wg, bg.reshape(1, E))


@functools.partial(
    pl.kernel, mesh=_MESH, compiler_params=_SC_PARAMS,
    out_type=jax.ShapeDtypeStruct((NW, 16), jnp.int32),
    scratch_types=[
        pltpu.VMEM((PW,), jnp.int32),
        pltpu.VMEM((16,), jnp.int32),
    ],
)
def _hist_kernel(e_hbm, counts_hbm, ids_v, hist_v):
    w = _wid()
    pltpu.sync_copy(e_hbm.at[pl.ds(w * PW, PW)], ids_v)
    lanes = _lanes()
    zeros16 = jnp.zeros((16,), jnp.int32)
    ones16 = jnp.ones((16,), jnp.int32)
    cnt = [zeros16] * E
    for j in range(PW // 16):
        ids = ids_v[pl.ds(j * 16, 16)]
        for e in range(E):
            m = ids == jnp.full((16,), e, jnp.int32)
            cnt[e] = cnt[e] + plsc.all_reduce_population_count(m)
    hist = zeros16
    for e in range(E):
        hist = jnp.where(lanes == jnp.full((16,), e, jnp.int32),
                         cnt[e], hist)
    hist_v[...] = hist
    pltpu.sync_copy(hist_v, counts_hbm.at[w])


@functools.partial(
    pl.kernel, mesh=_MESH, compiler_params=_SC_PARAMS,
    out_type=[
        jax.ShapeDtypeStruct((PPAD, D), jnp.float32),
        jax.ShapeDtypeStruct((PPAD,), jnp.float32),
        jax.ShapeDtypeStruct((P,), jnp.int32),
        jax.ShapeDtypeStruct((NBLK_PAD,), jnp.int32),
    ],
    scratch_types=[
        pltpu.VMEM((NW, 16), jnp.int32),
        pltpu.VMEM((PW,), jnp.int32),
        pltpu.VMEM((PW,), jnp.float32),
        pltpu.VMEM((PW,), jnp.int32),
        pltpu.VMEM((PW, D), jnp.float32),
        pltpu.VMEM((NBLK_PAD,), jnp.int32),
        pltpu.SemaphoreType.DMA,
        pltpu.SemaphoreType.DMA,
        pltpu.SemaphoreType.DMA,
        pltpu.SemaphoreType.DMA,
    ],
)
def _place_kernel(e_hbm, pw_hbm, counts_hbm, x_hbm,
                  xs_hbm, wsort_hbm, pairpos_hbm, bexp_hbm,
                  counts_v, ids_v, wv_v, pos_v, xrows_v, be_v,
                  sem, sem2, sem3, sem4):
    w = _wid()
    tbase = pl.multiple_of((w * PW) & (T - 1), PW)
    xcp = pltpu.async_copy(x_hbm.at[pl.ds(tbase, PW)], xrows_v, sem)
    ccp = pltpu.async_copy(counts_hbm, counts_v, sem2)
    icp = pltpu.async_copy(e_hbm.at[pl.ds(w * PW, PW)], ids_v, sem3)
    wcp = pltpu.async_copy(pw_hbm.at[pl.ds(w * PW, PW)], wv_v, sem4)
    ccp.wait()
    icp.wait()
    wcp.wait()

    def bcast(s):
        return jnp.full((16,), s, jnp.int32)

    zeros16 = jnp.zeros((16,), jnp.int32)
    ones16 = jnp.ones((16,), jnp.int32)
    lanes = _lanes()
    wv16 = bcast(w)

    totals = zeros16
    prefix = zeros16
    for wp in range(NW):
        row = counts_v[wp, :]
        totals = totals + row
        prefix = prefix + jnp.where(bcast(wp) < wv16, row, zeros16)

    cpad = ((totals + bcast(MB - 1)) // bcast(MB)) * bcast(MB)
    ends = plsc.cumsum(cpad)
    gpad = ends - cpad
    start = gpad + prefix

    cnt = [bcast(jnp.sum(jnp.where(lanes == bcast(e), start, zeros16)))
           for e in range(E)]

    for j in range(PW // 16):
        ids = ids_v[pl.ds(j * 16, 16)]
        pos = zeros16
        for e in range(E):
            m = ids == bcast(e)
            mi = jnp.where(m, ones16, zeros16)
            rank = plsc.cumsum(mi) - ones16
            pos = jnp.where(m, cnt[e] + rank, pos)
            cnt[e] = cnt[e] + plsc.all_reduce_population_count(m)
        pos_v[pl.ds(j * 16, 16)] = pos

    xcp.wait()
    ocp1 = pltpu.async_copy(xrows_v, xs_hbm.at[pos_v], sem)
    ocp2 = pltpu.async_copy(wv_v, wsort_hbm.at[pos_v], sem2)
    ocp3 = pltpu.async_copy(pos_v, pairpos_hbm.at[pl.ds(w * PW, PW)], sem3)
    ocp1.wait()
    ocp2.wait()
    ocp3.wait()

    @pl.when(w == 0)
    def _():
        nused = bcast(jnp.sum(jnp.where(lanes == bcast(E - 1), ends,
                                        zeros16))) // bcast(MB)
        for j in range(NBLK_PAD // 16):
            bstart = (bcast(j * 16) + lanes) * bcast(MB)
            be = zeros16
            for e in range(E - 1):
                end_e = bcast(jnp.sum(jnp.where(lanes == bcast(e), ends,
                                                zeros16)))
                be = be + jnp.where(bstart >= end_e, ones16, zeros16)
            if j == NBLK_PAD // 16 - 1:
                be = jnp.where(lanes == bcast(15), nused, be)
            be_v[pl.ds(j * 16, 16)] = be
        pltpu.sync_copy(be_v, bexp_hbm)


def _gelu_exact(v):
    return 0.5 * v * (1.0 + jax.lax.erf(v * (1.0 / math.sqrt(2.0))))


def _mlp_body(be_ref, xs_ref, w1_ref, b1_ref, w2_ref, b2_ref, ws_ref, y_ref,
              w1f_s, w2f_s, w1bf_s, w2bf_s, sems):
    b = pl.program_id(0)
    nused = be_ref[NBLK_PAD - 1]

    isb = []
    ords = []
    o = jnp.int32(-1)
    for bp in range(NBLK):
        first = jnp.logical_and(
            bp < nused,
            (be_ref[bp] != be_ref[bp - 1]) if bp else jnp.bool_(True))
        isb.append(first)
        o = o + first.astype(jnp.int32)
        ords.append(o)

    def sel(vals, idx):
        acc = jnp.int32(0)
        for bp in range(NBLK):
            acc = jnp.where(idx == bp, vals[bp].astype(jnp.int32), acc)
        return acc

    ordb = sel(ords, b)
    new_exp = sel(isb, b) > 0

    def group_expert(m):
        ge = jnp.int32(0)
        ex = jnp.bool_(False)
        for bp in range(NBLK):
            hit = jnp.logical_and(isb[bp], ords[bp] == m)
            ge = jnp.where(hit, be_ref[bp], ge)
            ex = jnp.logical_or(ex, hit)
        return ge, ex

    def fetch(m, slot):
        ge, ex = group_expert(m)

        @pl.when(ex)
        def _():
            pltpu.make_async_copy(w1_ref.at[ge], w1f_s.at[slot],
                                  sems.at[slot]).start()
            pltpu.make_async_copy(w2_ref.at[ge], w2f_s.at[slot],
                                  sems.at[slot]).start()

    @pl.when(b == 0)
    def _():
        fetch(jnp.int32(0), jnp.int32(0))
        fetch(jnp.int32(1), jnp.int32(1))

    @pl.when(jnp.logical_and(b < nused, new_exp))
    def _():
        slot = jnp.remainder(ordb, 2)
        e_cur = be_ref[b]
        pltpu.make_async_copy(w1_ref.at[e_cur], w1f_s.at[slot],
                              sems.at[slot]).wait()
        pltpu.make_async_copy(w2_ref.at[e_cur], w2f_s.at[slot],
                              sems.at[slot]).wait()
        w1bf_s[...] = w1f_s[slot].astype(jnp.bfloat16)
        w2bf_s[...] = w2f_s[slot].astype(jnp.bfloat16)
        fetch(ordb + 2, slot)

    @pl.when(b < nused)
    def _():
        xb = xs_ref[...].astype(jnp.bfloat16)
        h = jnp.dot(xb, w1bf_s[...], preferred_element_type=jnp.float32)
        h = _gelu_exact(h + b1_ref[0])
        y = jnp.dot(h.astype(jnp.bfloat16), w2bf_s[...],
                    preferred_element_type=jnp.float32) + b2_ref[0]
        y_ref[...] = ws_ref[...] * y


def _grouped_mlp(block_expert, Xs, W1, b1, W2, b2, w_sorted):
    grid_spec = pltpu.PrefetchScalarGridSpec(
        num_scalar_prefetch=1,
        grid=(NBLK,),
        in_specs=[
            pl.BlockSpec((MB, D), lambda b, be: (b, 0)),
            pl.BlockSpec(memory_space=pl.ANY),
            pl.BlockSpec((1, 1, H), lambda b, be: (be[b], 0, 0)),
            pl.BlockSpec(memory_space=pl.ANY),
            pl.BlockSpec((1, 1, D), lambda b, be: (be[b], 0, 0)),
            pl.BlockSpec((MB, 1), lambda b, be: (b, 0)),
        ],
        out_specs=pl.BlockSpec((MB, D), lambda b, be: (b, 0)),
        scratch_shapes=[
            pltpu.VMEM((2, D, H), jnp.float32),
            pltpu.VMEM((2, H, D), jnp.float32),
            pltpu.VMEM((D, H), jnp.bfloat16),
            pltpu.VMEM((H, D), jnp.bfloat16),
            pltpu.SemaphoreType.DMA((2,)),
        ],
    )
    return pl.pallas_call(
        _mlp_body,
        grid_spec=grid_spec,
        out_shape=jax.ShapeDtypeStruct((PPAD, D), jnp.float32),
    )(block_expert, Xs, W1, b1.reshape(E, 1, H), W2,
      b2.reshape(E, 1, D), w_sorted.reshape(PPAD, 1))


@functools.partial(
    pl.kernel, mesh=_MESH, compiler_params=_SC_PARAMS,
    out_type=jax.ShapeDtypeStruct((T, D), jnp.float32),
    scratch_types=[
        pltpu.VMEM((TCH,), jnp.int32),
        pltpu.VMEM((TCH,), jnp.int32),
        pltpu.VMEM((TCH, D), jnp.float32),
        pltpu.VMEM((TCH, D), jnp.float32),
        pltpu.SemaphoreType.DMA,
        pltpu.SemaphoreType.DMA,
    ],
)
def _combine_kernel(y_hbm, pairpos_hbm, out_hbm, idx0_v, idx1_v, buf0, buf1,
                    sem0, sem1):
    w = _wid()
    for c in range(TW // TCH):
        tbase = w * TW + c * TCH
        i0 = pltpu.async_copy(pairpos_hbm.at[pl.ds(tbase, TCH)], idx0_v, sem0)
        i1 = pltpu.async_copy(pairpos_hbm.at[pl.ds(T + tbase, TCH)], idx1_v,
                              sem1)
        i0.wait()
        i1.wait()
        cp0 = pltpu.async_copy(y_hbm.at[idx0_v], buf0, sem0)
        cp1 = pltpu.async_copy(y_hbm.at[idx1_v], buf1, sem1)
        cp0.wait()
        cp1.wait()

        def add_row(i, carry):
            for cc in range(D // 16):
                a = buf0[i, pl.ds(cc * 16, 16)]
                b = buf1[i, pl.ds(cc * 16, 16)]
                buf0[i, pl.ds(cc * 16, 16)] = a + b
            return carry

        lax.fori_loop(0, TCH, add_row, 0)
        pltpu.sync_copy(buf0, out_hbm.at[pl.ds(tbase, TCH)])


@jax.jit
def kernel(x, Wg, bg, W1, b1, W2, b2):
    x_flat = x.reshape(T, D)
    idx, pw = _gate(x_flat, Wg, bg)
    e_flat = idx.T.reshape(P)
    pw_flat = pw.T.reshape(P)

    counts = _hist_kernel(e_flat)
    Xs, w_sorted, pair_pos, block_expert = _place_kernel(
        e_flat, pw_flat, counts, x_flat)
    Y = _grouped_mlp(block_expert, Xs, W1, b1, W2, b2, w_sorted)
    out = _combine_kernel(Y, pair_pos)
    return out.reshape(B, S, D)


def _jnp_route(e_flat, pw_flat):
    c = jnp.bincount(e_flat, length=E)
    cpad = ((c + MB - 1) // MB) * MB
    ends = jnp.cumsum(cpad)
    gpad = ends - cpad
    start = jnp.cumsum(c) - c
    order = jnp.argsort(e_flat, stable=True)
    es = e_flat[order]
    pos_sorted = gpad[es] + (jnp.arange(P, dtype=jnp.int32) - start[es])
    pair_pos = jnp.zeros((P,), jnp.int32).at[order].set(pos_sorted)
    src_token = jnp.zeros((PPAD,), jnp.int32).at[pos_sorted].set(
        order.astype(jnp.int32) & (T - 1))
    w_sorted = jnp.zeros((PPAD,), jnp.float32).at[pos_sorted].set(
        pw_flat[order])
    block_expert = jnp.minimum(
        jnp.searchsorted(ends, jnp.arange(NBLK, dtype=jnp.int32) * MB,
                         side="right"), E - 1).astype(jnp.int32)
    return src_token, w_sorted, pair_pos, block_expert
